# baseline (speedup 1.0000x reference)
import functools

import jax
import jax.numpy as jnp
from jax import lax
from jax.experimental import pallas as pl
from jax.experimental.pallas import tpu as pltpu
import jax.experimental.pallas.tpu_sc as plsc

B = 4096
L = 50
D = 128
HID = 128
NCLS = 1000
THRESH = 0.15

NC, NS = 2, 16
NW = NC * NS
BPC = 2
ROWS = BPC * L
NLANE = 8

B0 = 3584
B1 = B - B0
MLP_BLK = 512


def _make_pool_body(base_batch, batches):
    bpw = batches // NW
    chunks = bpw // BPC
    base_row = base_batch // BPC

    def _pool_body(x_hbm, table_hbm, out_hbm, idx_v, rows_v, acc_v, sem0, sem1):
        cid = lax.axis_index("c")
        sid = lax.axis_index("s")
        wid = sid * NC + cid

        pltpu.sync_copy(x_hbm.at[pl.ds(base_row + wid * chunks, chunks)], idx_v)

        sems = (sem0, sem1)

        def start_gather(chunk, buf):
            chunk = jnp.minimum(chunk, chunks - 1)
            pltpu.async_copy(
                table_hbm.at[idx_v.at[chunk]], rows_v.at[buf], sems[buf])

        def wait_gather(buf):
            pltpu.make_async_copy(
                table_hbm.at[idx_v.at[0]], rows_v.at[buf], sems[buf]).wait()

        def reduce_chunk(chunk, buf):
            def body(r, accs):
                new = []
                for j in range(BPC):
                    for k in range(NLANE):
                        new.append(accs[j * NLANE + k]
                                   + rows_v[buf, j * L + r, pl.ds(k * 16, 16)])
                return tuple(new)

            init = tuple(jnp.zeros((16,), jnp.float32)
                         for _ in range(BPC * NLANE))
            accs = lax.fori_loop(0, L, body, init)
            for j in range(BPC):
                row = chunk * BPC + j
                for k in range(NLANE):
                    acc_v[row, pl.ds(k * 16, 16)] = accs[j * NLANE + k]

        start_gather(jnp.int32(0), 0)

        def outer(g, _):
            c0 = g * 2
            start_gather(c0 + 1, 1)
            wait_gather(0)
            reduce_chunk(c0, 0)
            start_gather(c0 + 2, 0)
            wait_gather(1)
            reduce_chunk(c0 + 1, 1)
            return 0

        lax.fori_loop(0, chunks // 2, outer, 0)
        wait_gather(0)

        pltpu.sync_copy(acc_v, out_hbm.at[pl.ds(wid * bpw, bpw)])

    return _pool_body


def _pool(base_batch, batches, x2, emb_table):
    bpw = batches // NW
    chunks = bpw // BPC
    mesh = plsc.VectorSubcoreMesh(core_axis_name="c", subcore_axis_name="s",
                                  num_cores=NC, num_subcores=NS)
    return pl.kernel(
        _make_pool_body(base_batch, batches),
        out_type=jax.ShapeDtypeStruct((batches, D), jnp.float32),
        mesh=mesh,
        scratch_types=[
            pltpu.VMEM((chunks, ROWS), jnp.int32),
            pltpu.VMEM((2, ROWS, D), jnp.float32),
            pltpu.VMEM((bpw, D), jnp.float32),
            pltpu.SemaphoreType.DMA,
            pltpu.SemaphoreType.DMA,
        ],
        name=f"pool_b{base_batch}",
    )(x2, emb_table)


def _mlp_first_body(x_ref, w1_ref, b1_ref, w2t_ref, b2t_ref, ot_ref):
    xm = x_ref[...] / jnp.float32(L)
    h = jnp.dot(xm, w1_ref[...], preferred_element_type=jnp.float32)
    h = h + b1_ref[...]
    h = jnp.where(h >= THRESH, h, 0.0)
    ot = lax.dot_general(w2t_ref[...], h, (((1,), (1,)), ((), ())),
                         preferred_element_type=jnp.float32)
    ot_ref[...] = ot + b2t_ref[...]


def _mlp_alias_body(prev_ref, x_ref, w1_ref, b1_ref, w2t_ref, b2t_ref, ot_ref):
    del prev_ref
    _mlp_first_body(x_ref, w1_ref, b1_ref, w2t_ref, b2t_ref, ot_ref)


def _mlp_piece(col0_blocks, nblk, out_prev, pooled, w1, b1, w2t, b2t):
    data_specs = [
        pl.BlockSpec((MLP_BLK, D), lambda i: (i, 0)),
        pl.BlockSpec((D, HID), lambda i: (0, 0)),
        pl.BlockSpec((1, HID), lambda i: (0, 0)),
        pl.BlockSpec((NCLS, HID), lambda i: (0, 0)),
        pl.BlockSpec((NCLS, 1), lambda i: (0, 0)),
    ]
    out_spec = pl.BlockSpec((NCLS, MLP_BLK), lambda i: (0, col0_blocks + i))
    out_shape = jax.ShapeDtypeStruct((NCLS, B), jnp.float32)
    if out_prev is None:
        return pl.pallas_call(
            _mlp_first_body,
            grid=(nblk,),
            in_specs=data_specs,
            out_specs=out_spec,
            out_shape=out_shape,
            name="mlp_big",
        )(pooled, w1, b1, w2t, b2t)
    return pl.pallas_call(
        _mlp_alias_body,
        grid=(nblk,),
        in_specs=[pl.BlockSpec((8, 128), lambda i: (0, 0))] + data_specs,
        out_specs=out_spec,
        out_shape=out_shape,
        input_output_aliases={0: 0},
        name="mlp_small",
    )(out_prev, pooled, w1, b1, w2t, b2t)


def kernel(x, emb_table, W1, b1, W2, b2):
    x2 = x.reshape(B * L // ROWS, ROWS)
    b1r = b1.reshape(1, HID)
    b2t = b2.reshape(NCLS, 1)
    w2t = W2.T
    pooled0 = _pool(0, B0, x2, emb_table)
    pooled1 = _pool(B0, B1, x2, emb_table)
    out_t = _mlp_piece(0, B0 // MLP_BLK, None, pooled0, W1, b1r, w2t, b2t)
    out_t = _mlp_piece(B0 // MLP_BLK, B1 // MLP_BLK, out_t, pooled1,
                       W1, b1r, w2t, b2t)
    return out_t.T


# speedup vs baseline: 1.0138x; 1.0138x over previous
import functools

import jax
import jax.numpy as jnp
from jax import lax
from jax.experimental import pallas as pl
from jax.experimental.pallas import tpu as pltpu
import jax.experimental.pallas.tpu_sc as plsc

B = 4096
L = 50
D = 128
HID = 128
NCLS = 1000
THRESH = 0.15

NC, NS = 2, 16
NW = NC * NS
BPW = B // NW
BPC = 2
ROWS = BPC * L
CHUNKS = BPW // BPC
NLANE = 8


def _pool_body(x_hbm, table_hbm, out_hbm, idx_v, rows_v, acc_v, sem0, sem1):
    cid = lax.axis_index("c")
    sid = lax.axis_index("s")
    wid = sid * NC + cid

    pltpu.sync_copy(x_hbm.at[pl.ds(wid * CHUNKS, CHUNKS)], idx_v)

    sems = (sem0, sem1)

    def start_gather(chunk, buf):
        chunk = jnp.minimum(chunk, CHUNKS - 1)
        pltpu.async_copy(
            table_hbm.at[idx_v.at[chunk]], rows_v.at[buf], sems[buf])

    def wait_gather(buf):
        pltpu.make_async_copy(
            table_hbm.at[idx_v.at[0]], rows_v.at[buf], sems[buf]).wait()

    def reduce_chunk(chunk, buf):
        def body(r, accs):
            new = []
            for j in range(BPC):
                for k in range(NLANE):
                    new.append(accs[j * NLANE + k]
                               + rows_v[buf, j * L + r, pl.ds(k * 16, 16)])
            return tuple(new)

        init = tuple(jnp.zeros((16,), jnp.float32) for _ in range(BPC * NLANE))
        accs = lax.fori_loop(0, L, body, init)
        for j in range(BPC):
            row = chunk * BPC + j
            for k in range(NLANE):
                acc_v[row, pl.ds(k * 16, 16)] = accs[j * NLANE + k]

    start_gather(jnp.int32(0), 0)

    def outer(g, _):
        c0 = g * 2
        start_gather(c0 + 1, 1)
        wait_gather(0)
        reduce_chunk(c0, 0)
        start_gather(c0 + 2, 0)
        wait_gather(1)
        reduce_chunk(c0 + 1, 1)
        return 0

    lax.fori_loop(0, CHUNKS // 2, outer, 0)
    wait_gather(0)

    pltpu.sync_copy(acc_v, out_hbm.at[pl.ds(wid * BPW, BPW)])


@functools.partial(jax.jit, static_argnames=())
def _pool(x2, emb_table):
    mesh = plsc.VectorSubcoreMesh(core_axis_name="c", subcore_axis_name="s",
                                  num_cores=NC, num_subcores=NS)
    return pl.kernel(
        _pool_body,
        out_type=jax.ShapeDtypeStruct((B, D), jnp.float32),
        mesh=mesh,
        scratch_types=[
            pltpu.VMEM((CHUNKS, ROWS), jnp.int32),
            pltpu.VMEM((2, ROWS, D), jnp.float32),
            pltpu.VMEM((BPW, D), jnp.float32),
            pltpu.SemaphoreType.DMA,
            pltpu.SemaphoreType.DMA,
        ],
    )(x2, emb_table)


def _mlp_body(x_ref, w1_ref, b1_ref, w2t_ref, b2t_ref, ot_ref):
    xm = x_ref[...] / jnp.float32(L)
    h = jnp.dot(xm, w1_ref[...], preferred_element_type=jnp.float32)
    h = h + b1_ref[...]
    h = jnp.where(h >= THRESH, h, 0.0)
    ot = lax.dot_general(w2t_ref[...], h, (((1,), (1,)), ((), ())),
                         preferred_element_type=jnp.float32)
    ot_ref[...] = ot + b2t_ref[...]


def _mlp(pooled, w1, b1, w2t, b2t):
    blk = 512
    return pl.pallas_call(
        _mlp_body,
        grid=(B // blk,),
        in_specs=[
            pl.BlockSpec((blk, D), lambda i: (i, 0)),
            pl.BlockSpec((D, HID), lambda i: (0, 0)),
            pl.BlockSpec((1, HID), lambda i: (0, 0)),
            pl.BlockSpec((NCLS, HID), lambda i: (0, 0)),
            pl.BlockSpec((NCLS, 1), lambda i: (0, 0)),
        ],
        out_specs=pl.BlockSpec((NCLS, blk), lambda i: (0, i)),
        out_shape=jax.ShapeDtypeStruct((NCLS, B), jnp.float32),
    )(pooled, w1, b1, w2t, b2t)


def kernel(x, emb_table, W1, b1, W2, b2):
    x2 = x.reshape(NW * CHUNKS, ROWS)
    pooled = _pool(x2, emb_table)
    out_t = _mlp(pooled, W1, b1.reshape(1, HID), W2.T, b2.reshape(NCLS, 1))
    return out_t.T
